# baseline (speedup 1.0000x reference)
import functools

import jax
import jax.numpy as jnp
from jax.experimental import pallas as pl

N_EXPERTS = 64
TOP_K = 8
BLOCK_ROWS = 256


def _gate_body(x_ref, w_ref, idx_ref, wgt_ref):
    x = x_ref[...]
    w = w_ref[...]
    logits = jax.lax.dot_general(
        x, w, (((1,), (1,)), ((), ())), preferred_element_type=jnp.float32
    )
    m = jnp.max(logits, axis=1, keepdims=True)
    e = jnp.exp(logits - m)
    s = jnp.sum(e, axis=1, keepdims=True)
    scores = e / s

    col = jax.lax.broadcasted_iota(jnp.int32, scores.shape, 1)
    neg_inf = jnp.float32(-jnp.inf)
    idxs = []
    vals = []
    sc = scores
    for _ in range(TOP_K):
        mk = jnp.max(sc, axis=1, keepdims=True)
        ak = jnp.min(jnp.where(sc == mk, col, N_EXPERTS), axis=1, keepdims=True)
        vals.append(mk)
        idxs.append(ak)
        sc = jnp.where(col == ak, neg_inf, sc)
    idx_ref[...] = jnp.concatenate(idxs, axis=1)
    wgt_ref[...] = jnp.concatenate(vals, axis=1)


@jax.jit
def kernel(hidden_states, weight):
    bsz, seq_len, h = hidden_states.shape
    n = bsz * seq_len
    x = hidden_states.reshape(n, h)
    grid = (n // BLOCK_ROWS,)
    idx, wgt = pl.pallas_call(
        _gate_body,
        grid=grid,
        in_specs=[
            pl.BlockSpec((BLOCK_ROWS, h), lambda i: (i, 0)),
            pl.BlockSpec((N_EXPERTS, h), lambda i: (0, 0)),
        ],
        out_specs=[
            pl.BlockSpec((BLOCK_ROWS, TOP_K), lambda i: (i, 0)),
            pl.BlockSpec((BLOCK_ROWS, TOP_K), lambda i: (i, 0)),
        ],
        out_shape=[
            jax.ShapeDtypeStruct((n, TOP_K), jnp.int32),
            jax.ShapeDtypeStruct((n, TOP_K), jnp.float32),
        ],
    )(x, weight)
    return idx, wgt


# speedup vs baseline: 2.1963x; 2.1963x over previous
import jax
import jax.numpy as jnp
from jax.experimental import pallas as pl

N_EXPERTS = 64
TOP_K = 8
BLOCK_ROWS = 512


def _gate_body(x_ref, w_ref, idx_ref, wgt_ref):
    x = x_ref[...]
    w = w_ref[...]
    logits = jax.lax.dot_general(
        x, w, (((1,), (1,)), ((), ())), preferred_element_type=jnp.float32
    )
    lt = logits.T
    m = jnp.max(lt, axis=0, keepdims=True)
    e = jnp.exp(lt - m)
    rs = 1.0 / jnp.sum(e, axis=0, keepdims=True)

    row = jax.lax.broadcasted_iota(jnp.int32, lt.shape, 0).astype(jnp.float32)
    neg_inf = jnp.float32(-jnp.inf)
    idxs = []
    vals = []
    sc = lt
    for _ in range(TOP_K):
        mk = jnp.max(sc, axis=0, keepdims=True)
        ak = jnp.min(
            jnp.where(sc == mk, row, jnp.float32(N_EXPERTS)), axis=0, keepdims=True
        )
        vals.append(jnp.exp(mk - m) * rs)
        idxs.append(ak)
        sc = jnp.where(row == ak, neg_inf, sc)
    idx_ref[...] = jnp.concatenate(idxs, axis=0).astype(jnp.int32)
    wgt_ref[...] = jnp.concatenate(vals, axis=0)


@jax.jit
def kernel(hidden_states, weight):
    bsz, seq_len, h = hidden_states.shape
    n = bsz * seq_len
    x = hidden_states.reshape(n, h)
    grid = (n // BLOCK_ROWS,)
    idx_t, wgt_t = pl.pallas_call(
        _gate_body,
        grid=grid,
        in_specs=[
            pl.BlockSpec((BLOCK_ROWS, h), lambda i: (i, 0)),
            pl.BlockSpec((N_EXPERTS, h), lambda i: (0, 0)),
        ],
        out_specs=[
            pl.BlockSpec((TOP_K, BLOCK_ROWS), lambda i: (0, i)),
            pl.BlockSpec((TOP_K, BLOCK_ROWS), lambda i: (0, i)),
        ],
        out_shape=[
            jax.ShapeDtypeStruct((TOP_K, n), jnp.int32),
            jax.ShapeDtypeStruct((TOP_K, n), jnp.float32),
        ],
    )(x, weight)
    return idx_t.T, wgt_t.T


# speedup vs baseline: 2.2803x; 1.0383x over previous
import jax
import jax.numpy as jnp
from jax.experimental import pallas as pl

N_EXPERTS = 64
TOP_K = 8
BLOCK_ROWS = 1024


def _gate_body(x_ref, w_ref, idx_ref, wgt_ref):
    x = x_ref[...]
    w = w_ref[...]
    logits = jax.lax.dot_general(
        x, w, (((1,), (1,)), ((), ())), preferred_element_type=jnp.float32
    )
    lt = logits.T
    m = jnp.max(lt, axis=0, keepdims=True)
    e = jnp.exp(lt - m)
    rs = 1.0 / jnp.sum(e, axis=0, keepdims=True)

    row = jax.lax.broadcasted_iota(jnp.int32, lt.shape, 0).astype(jnp.float32)
    neg_inf = jnp.float32(-jnp.inf)
    idxs = []
    vals = []
    sc = lt
    for _ in range(TOP_K):
        mk = jnp.max(sc, axis=0, keepdims=True)
        ak = jnp.min(
            jnp.where(sc == mk, row, jnp.float32(N_EXPERTS)), axis=0, keepdims=True
        )
        vals.append(jnp.exp(mk - m) * rs)
        idxs.append(ak)
        sc = jnp.where(row == ak, neg_inf, sc)
    idx_ref[...] = jnp.concatenate(idxs, axis=0).astype(jnp.int32)
    wgt_ref[...] = jnp.concatenate(vals, axis=0)


@jax.jit
def kernel(hidden_states, weight):
    bsz, seq_len, h = hidden_states.shape
    n = bsz * seq_len
    x = hidden_states.reshape(n, h)
    grid = (n // BLOCK_ROWS,)
    idx_t, wgt_t = pl.pallas_call(
        _gate_body,
        grid=grid,
        in_specs=[
            pl.BlockSpec((BLOCK_ROWS, h), lambda i: (i, 0)),
            pl.BlockSpec((N_EXPERTS, h), lambda i: (0, 0)),
        ],
        out_specs=[
            pl.BlockSpec((TOP_K, BLOCK_ROWS), lambda i: (0, i)),
            pl.BlockSpec((TOP_K, BLOCK_ROWS), lambda i: (0, i)),
        ],
        out_shape=[
            jax.ShapeDtypeStruct((TOP_K, n), jnp.int32),
            jax.ShapeDtypeStruct((TOP_K, n), jnp.float32),
        ],
    )(x, weight)
    return idx_t.T, wgt_t.T
